# baseline (speedup 1.0000x reference)
import jax
import jax.numpy as jnp
from jax import lax
from jax.experimental import pallas as pl
from jax.experimental.pallas import tpu as pltpu
from jax.experimental.pallas import tpu_sc as plsc

B = 16384
F = 26
D = 16
NW = 32
E = B // NW
C = 64
NCH = E // C
RPC = F * C
IW = 128
NG = RPC // IW


def _fm_body(x_r, emb, fc, wv, bv, out, idx2, rows2, fcv2, outv, tbuf, pv, sems):
    wid = lax.axis_index("c") * 16 + lax.axis_index("s")

    pltpu.sync_copy(wv, pv.at[0])
    pltpu.sync_copy(bv, pv.at[1])

    def fire(g, b):
        row0 = (wid * NCH + g) * NG
        pltpu.sync_copy(x_r.at[pl.ds(row0, NG)], idx2.at[b])
        for r in range(NG):
            pltpu.async_copy(emb.at[idx2.at[b, r]],
                             rows2.at[b, pl.ds(r * IW, IW)], sems.at[b])
            pltpu.async_copy(fc.at[idx2.at[b, r]],
                             fcv2.at[b, pl.ds(r * IW, IW)], sems.at[b])

    def drain(b):
        for r in range(NG):
            pltpu.make_async_copy(emb.at[idx2.at[b, r]],
                                  rows2.at[b, pl.ds(r * IW, IW)],
                                  sems.at[b]).wait()
            pltpu.make_async_copy(fc.at[idx2.at[b, r]],
                                  fcv2.at[b, pl.ds(r * IW, IW)],
                                  sems.at[b]).wait()

    lane = lax.iota(jnp.int32, 16)
    lane16 = lane * 16
    mask10 = (lane < (F - 16)).astype(jnp.float32)

    def compute(g, b):
        wvec = pv[0, :]
        bvec = pv[1, :]
        for grp in range(C // 16):
            @pl.loop(0, 16)
            def _elem(i):
                e = grp * 16 + i
                r0 = e * F
                s = jnp.zeros((16,), jnp.float32)
                q = jnp.zeros((16,), jnp.float32)
                for f in range(F):
                    v = rows2[b, r0 + f, :]
                    s = s + v
                    q = q + v * v
                f1 = fcv2[b, pl.ds(r0, 16)]
                f2 = fcv2[b, pl.ds(r0 + 16, 16)]
                t = 0.5 * (s * s - q) + wvec * (f1 + f2 * mask10)
                tbuf[pl.ds(i * 16, 16)] = t

            acc = jnp.zeros((16,), jnp.float32)
            for dcol in range(16):
                acc = acc + plsc.load_gather(tbuf, [lane16 + dcol])
            tot = acc + bvec
            outv[pl.ds(grp * 16, 16)] = 1.0 / (1.0 + jnp.exp(-tot))
        pltpu.sync_copy(outv, out.at[pl.ds(wid * E + g * C, C)])

    fire(0, 0)

    @pl.loop(0, NCH, step=2)
    def _chunks(gg):
        fire(gg + 1, 1)
        drain(0)
        compute(gg, 0)

        @pl.when(gg + 2 < NCH)
        def _refill():
            fire(gg + 2, 0)

        drain(1)
        compute(gg + 1, 1)


def kernel(x, emb_table, fc_table, lin_w, lin_b):
    x_r = x.astype(jnp.int32).reshape(B * F // IW, IW)
    fc_flat = fc_table.reshape(-1)
    wv = jnp.broadcast_to(lin_w.reshape(()), (16,)).astype(jnp.float32)
    bv = jnp.broadcast_to(lin_b.reshape(()), (16,)).astype(jnp.float32)

    mesh = plsc.VectorSubcoreMesh(core_axis_name="c", subcore_axis_name="s")
    out = pl.kernel(
        _fm_body,
        out_type=jax.ShapeDtypeStruct((B,), jnp.float32),
        mesh=mesh,
        compiler_params=pltpu.CompilerParams(needs_layout_passes=False,
                                             use_tc_tiling_on_sc=False),
        scratch_types=[
            pltpu.VMEM((2, NG, IW), jnp.int32),
            pltpu.VMEM((2, RPC, D), jnp.float32),
            pltpu.VMEM((2, RPC + 16), jnp.float32),
            pltpu.VMEM((C,), jnp.float32),
            pltpu.VMEM((256,), jnp.float32),
            pltpu.VMEM((2, 16), jnp.float32),
            pltpu.SemaphoreType.DMA((2,)),
        ],
    )(x_r, emb_table, fc_flat, wv, bv)
    return out.reshape(B, 1)


# speedup vs baseline: 1.0514x; 1.0514x over previous
import jax
import jax.numpy as jnp
from jax import lax
from jax.experimental import pallas as pl
from jax.experimental.pallas import tpu as pltpu
from jax.experimental.pallas import tpu_sc as plsc

B = 16384
F = 26
D = 16
NW = 32
E = B // NW
C = 64
NCH = E // C
RPC = F * C
IW = 128
NG = RPC // IW

NR = 1000012
TB = NR // 128
TAILR = NR - TB * 128
TBQ, TBR = divmod(TB, NW)


def _tr_body(emb_t, tail, out, bufa, bufb, obufa, obufb, tvbuf, sems):
    wid = lax.axis_index("c") * 16 + lax.axis_index("s")
    base = wid * TBQ + jnp.minimum(wid, TBR)
    cnt = TBQ + (wid < TBR).astype(jnp.int32)
    lane = lax.iota(jnp.int32, 16)

    def fire_in(blk, buf, s):
        pltpu.async_copy(emb_t.at[:, pl.ds(blk * 128, 128)], buf, sems.at[s])

    def wait_in(buf, s):
        pltpu.make_async_copy(emb_t.at[:, pl.ds(0, 128)], buf,
                              sems.at[s]).wait()

    def compute(blk, buf, obuf):
        @pl.loop(0, 128, unroll=8)
        def _col(j):
            jv = jnp.zeros((16,), jnp.int32) + j
            col = plsc.load_gather(buf, [lane, jv])
            obuf[pl.ds(j * 16, 16)] = col

        pltpu.sync_copy(obuf, out.at[pl.ds(blk * 2048, 2048)])

    fire_in(base, bufa, 0)
    last = base + cnt

    @pl.loop(0, (cnt + 1) // 2)
    def _blkpair(i):
        blk_a = base + 2 * i
        blk_b = blk_a + 1

        @pl.when(blk_b < last)
        def _fb():
            fire_in(blk_b, bufb, 1)

        wait_in(bufa, 0)
        compute(blk_a, bufa, obufa)

        @pl.when(blk_a + 2 < last)
        def _fa():
            fire_in(blk_a + 2, bufa, 0)

        @pl.when(blk_b < last)
        def _cb():
            wait_in(bufb, 1)
            compute(blk_b, bufb, obufb)

    @pl.when(wid == 0)
    def _tail():
        pltpu.sync_copy(tail, tvbuf)
        pltpu.sync_copy(tvbuf, out.at[pl.ds(TB * 2048, TAILR * D)])


def _fm_body(x_r, emb, fc, wv, bv, out, idx2, rows2, fcv2, outv, tbuf, pv, sems):
    wid = lax.axis_index("c") * 16 + lax.axis_index("s")

    pltpu.sync_copy(wv, pv.at[0])
    pltpu.sync_copy(bv, pv.at[1])

    def fire(g, b):
        row0 = (wid * NCH + g) * NG
        pltpu.sync_copy(x_r.at[pl.ds(row0, NG)], idx2.at[b])
        for r in range(NG):
            pltpu.async_copy(emb.at[idx2.at[b, r]],
                             rows2.at[b, pl.ds(r * IW, IW)], sems.at[b])
            pltpu.async_copy(fc.at[idx2.at[b, r]],
                             fcv2.at[b, pl.ds(r * IW, IW)], sems.at[b])

    def drain(b):
        for r in range(NG):
            pltpu.make_async_copy(emb.at[idx2.at[b, r]],
                                  rows2.at[b, pl.ds(r * IW, IW)],
                                  sems.at[b]).wait()
            pltpu.make_async_copy(fc.at[idx2.at[b, r]],
                                  fcv2.at[b, pl.ds(r * IW, IW)],
                                  sems.at[b]).wait()

    lane = lax.iota(jnp.int32, 16)
    lane16 = lane * 16
    mask10 = (lane < (F - 16)).astype(jnp.float32)

    def compute(g, b):
        wvec = pv[0, :]
        bvec = pv[1, :]
        for grp in range(C // 16):
            @pl.loop(0, 16)
            def _elem(i):
                e = grp * 16 + i
                r0 = e * F
                s = jnp.zeros((16,), jnp.float32)
                q = jnp.zeros((16,), jnp.float32)
                for f in range(F):
                    v = rows2[b, r0 + f, :]
                    s = s + v
                    q = q + v * v
                f1 = fcv2[b, pl.ds(r0, 16)]
                f2 = fcv2[b, pl.ds(r0 + 16, 16)]
                t = 0.5 * (s * s - q) + wvec * (f1 + f2 * mask10)
                tbuf[pl.ds(i * 16, 16)] = t

            acc = jnp.zeros((16,), jnp.float32)
            for dcol in range(16):
                acc = acc + plsc.load_gather(tbuf, [lane16 + dcol])
            tot = acc + bvec
            outv[pl.ds(grp * 16, 16)] = 1.0 / (1.0 + jnp.exp(-tot))
        pltpu.sync_copy(outv, out.at[pl.ds(wid * E + g * C, C)])

    fire(0, 0)

    @pl.loop(0, NCH, step=2)
    def _chunks(gg):
        fire(gg + 1, 1)
        drain(0)
        compute(gg, 0)

        @pl.when(gg + 2 < NCH)
        def _refill():
            fire(gg + 2, 0)

        drain(1)
        compute(gg + 1, 1)


def kernel(x, emb_table, fc_table, lin_w, lin_b):
    mesh = plsc.VectorSubcoreMesh(core_axis_name="c", subcore_axis_name="s")

    emb_t = emb_table.T
    tail = emb_table[TB * 128:, :].reshape(-1)
    emb_lin = pl.kernel(
        _tr_body,
        out_type=jax.ShapeDtypeStruct((NR * D,), jnp.float32),
        mesh=mesh,
        compiler_params=pltpu.CompilerParams(needs_layout_passes=False,
                                             use_tc_tiling_on_sc=True),
        scratch_types=[
            pltpu.VMEM((16, 128), jnp.float32),
            pltpu.VMEM((16, 128), jnp.float32),
            pltpu.VMEM((2048,), jnp.float32),
            pltpu.VMEM((2048,), jnp.float32),
            pltpu.VMEM((TAILR * D,), jnp.float32),
            pltpu.SemaphoreType.DMA((2,)),
        ],
    )(emb_t, tail)
    emb2 = emb_lin.reshape(NR, D)

    x_r = x.astype(jnp.int32).reshape(B * F // IW, IW)
    fc_flat = fc_table.reshape(-1)
    wv = jnp.broadcast_to(lin_w.reshape(()), (16,)).astype(jnp.float32)
    bv = jnp.broadcast_to(lin_b.reshape(()), (16,)).astype(jnp.float32)

    out = pl.kernel(
        _fm_body,
        out_type=jax.ShapeDtypeStruct((B,), jnp.float32),
        mesh=mesh,
        compiler_params=pltpu.CompilerParams(needs_layout_passes=False,
                                             use_tc_tiling_on_sc=False),
        scratch_types=[
            pltpu.VMEM((2, NG, IW), jnp.int32),
            pltpu.VMEM((2, RPC, D), jnp.float32),
            pltpu.VMEM((2, RPC + 16), jnp.float32),
            pltpu.VMEM((C,), jnp.float32),
            pltpu.VMEM((256,), jnp.float32),
            pltpu.VMEM((2, 16), jnp.float32),
            pltpu.SemaphoreType.DMA((2,)),
        ],
    )(x_r, emb2, fc_flat, wv, bv)
    return out.reshape(B, 1)


# speedup vs baseline: 2.8013x; 2.6644x over previous
import jax
import jax.numpy as jnp
from jax import lax
from jax.experimental import pallas as pl
from jax.experimental.pallas import tpu as pltpu
from jax.experimental.pallas import tpu_sc as plsc

B = 16384
F = 26
D = 16
NW = 32
E = B // NW
C = 64
NCH = E // C
RPC = F * C
IW = 128
NG = RPC // IW

NR = 1000012
GW = 512
TG = NR // GW
TAILR = NR - TG * GW
TGQ, TGR = divmod(TG, NW)


def _tr_body(emb_t, tail, out, bufa, bufb, obufa, obufb, tvbuf, sems):
    wid = lax.axis_index("c") * 16 + lax.axis_index("s")
    baseg = wid * TGQ + jnp.minimum(wid, TGR)
    cntg = TGQ + (wid < TGR).astype(jnp.int32)
    lastg = baseg + cntg
    lane16 = lax.iota(jnp.int32, 16) * 16

    def fire_in(g, buf, s):
        pltpu.async_copy(emb_t.at[:, pl.ds(g * GW, GW)], buf, sems.at[s])

    def wait_in(buf, s):
        pltpu.make_async_copy(emb_t.at[:, pl.ds(0, GW)], buf,
                              sems.at[s]).wait()

    def fire_out(g, obuf, s):
        pltpu.async_copy(obuf, out.at[pl.ds(g * (GW * D), GW * D)],
                         sems.at[s])

    def wait_out(obuf, s):
        pltpu.make_async_copy(obuf, out.at[pl.ds(0, GW * D)],
                              sems.at[s]).wait()

    def compute(buf, obuf):
        @pl.loop(0, GW // 16)
        def _c16(c):
            vec = lane16 + c * 256
            for d in range(D):
                v = buf[d, pl.ds(c * 16, 16)]
                plsc.store_scatter(obuf, [vec + d], v)

    fire_in(baseg, bufa, 0)

    @pl.loop(0, (cntg + 1) // 2)
    def _pair(i):
        g_a = baseg + 2 * i
        g_b = g_a + 1

        @pl.when(g_b < lastg)
        def _fb():
            fire_in(g_b, bufb, 1)

        wait_in(bufa, 0)

        @pl.when(i > 0)
        def _woa():
            wait_out(obufa, 2)

        compute(bufa, obufa)
        fire_out(g_a, obufa, 2)

        @pl.when(g_a + 2 < lastg)
        def _fa():
            fire_in(g_a + 2, bufa, 0)

        @pl.when(g_b < lastg)
        def _cb():
            wait_in(bufb, 1)

            @pl.when(i > 0)
            def _wob():
                wait_out(obufb, 3)

            compute(bufb, obufb)
            fire_out(g_b, obufb, 3)

    wait_out(obufa, 2)
    wait_out(obufb, 3)

    @pl.when(wid == 0)
    def _tail():
        pltpu.sync_copy(tail, tvbuf)
        pltpu.sync_copy(tvbuf, out.at[pl.ds(TG * GW * D, TAILR * D)])


def _fm_body(x_r, emb, fc, wv, bv, out, idx2, rows2, fcv2, outv, tbuf, pv, sems):
    wid = lax.axis_index("c") * 16 + lax.axis_index("s")

    pltpu.sync_copy(wv, pv.at[0])
    pltpu.sync_copy(bv, pv.at[1])

    def fire(g, b):
        row0 = (wid * NCH + g) * NG
        pltpu.sync_copy(x_r.at[pl.ds(row0, NG)], idx2.at[b])
        for r in range(NG):
            pltpu.async_copy(emb.at[idx2.at[b, r]],
                             rows2.at[b, pl.ds(r * IW, IW)], sems.at[b])
            pltpu.async_copy(fc.at[idx2.at[b, r]],
                             fcv2.at[b, pl.ds(r * IW, IW)], sems.at[b])

    def drain(b):
        for r in range(NG):
            pltpu.make_async_copy(emb.at[idx2.at[b, r]],
                                  rows2.at[b, pl.ds(r * IW, IW)],
                                  sems.at[b]).wait()
            pltpu.make_async_copy(fc.at[idx2.at[b, r]],
                                  fcv2.at[b, pl.ds(r * IW, IW)],
                                  sems.at[b]).wait()

    lane = lax.iota(jnp.int32, 16)
    lane16 = lane * 16
    mask10 = (lane < (F - 16)).astype(jnp.float32)

    def compute(g, b):
        wvec = pv[0, :]
        bvec = pv[1, :]
        for grp in range(C // 16):
            @pl.loop(0, 16)
            def _elem(i):
                e = grp * 16 + i
                r0 = e * F
                s = jnp.zeros((16,), jnp.float32)
                q = jnp.zeros((16,), jnp.float32)
                for f in range(F):
                    v = rows2[b, r0 + f, :]
                    s = s + v
                    q = q + v * v
                f1 = fcv2[b, pl.ds(r0, 16)]
                f2 = fcv2[b, pl.ds(r0 + 16, 16)]
                t = 0.5 * (s * s - q) + wvec * (f1 + f2 * mask10)
                tbuf[pl.ds(i * 16, 16)] = t

            acc = jnp.zeros((16,), jnp.float32)
            for dcol in range(16):
                acc = acc + plsc.load_gather(tbuf, [lane16 + dcol])
            tot = acc + bvec
            outv[pl.ds(grp * 16, 16)] = 1.0 / (1.0 + jnp.exp(-tot))
        pltpu.sync_copy(outv, out.at[pl.ds(wid * E + g * C, C)])

    fire(0, 0)

    @pl.loop(0, NCH, step=2)
    def _chunks(gg):
        fire(gg + 1, 1)
        drain(0)
        compute(gg, 0)

        @pl.when(gg + 2 < NCH)
        def _refill():
            fire(gg + 2, 0)

        drain(1)
        compute(gg + 1, 1)


def kernel(x, emb_table, fc_table, lin_w, lin_b):
    mesh = plsc.VectorSubcoreMesh(core_axis_name="c", subcore_axis_name="s")

    emb_t = emb_table.T
    tail = emb_table[TG * GW:, :].reshape(-1)
    emb_lin = pl.kernel(
        _tr_body,
        out_type=jax.ShapeDtypeStruct((NR * D,), jnp.float32),
        mesh=mesh,
        compiler_params=pltpu.CompilerParams(needs_layout_passes=False,
                                             use_tc_tiling_on_sc=True),
        scratch_types=[
            pltpu.VMEM((D, GW), jnp.float32),
            pltpu.VMEM((D, GW), jnp.float32),
            pltpu.VMEM((GW * D,), jnp.float32),
            pltpu.VMEM((GW * D,), jnp.float32),
            pltpu.VMEM((TAILR * D,), jnp.float32),
            pltpu.SemaphoreType.DMA((4,)),
        ],
    )(emb_t, tail)
    emb2 = emb_lin.reshape(NR, D)

    x_r = x.astype(jnp.int32).reshape(B * F // IW, IW)
    fc_flat = fc_table.reshape(-1)
    wv = jnp.broadcast_to(lin_w.reshape(()), (16,)).astype(jnp.float32)
    bv = jnp.broadcast_to(lin_b.reshape(()), (16,)).astype(jnp.float32)

    out = pl.kernel(
        _fm_body,
        out_type=jax.ShapeDtypeStruct((B,), jnp.float32),
        mesh=mesh,
        compiler_params=pltpu.CompilerParams(needs_layout_passes=False,
                                             use_tc_tiling_on_sc=False),
        scratch_types=[
            pltpu.VMEM((2, NG, IW), jnp.int32),
            pltpu.VMEM((2, RPC, D), jnp.float32),
            pltpu.VMEM((2, RPC + 16), jnp.float32),
            pltpu.VMEM((C,), jnp.float32),
            pltpu.VMEM((256,), jnp.float32),
            pltpu.VMEM((2, 16), jnp.float32),
            pltpu.SemaphoreType.DMA((2,)),
        ],
    )(x_r, emb2, fc_flat, wv, bv)
    return out.reshape(B, 1)
